# baseline (speedup 1.0000x reference)
import functools

import jax
import jax.numpy as jnp
from jax import lax
from jax.experimental import pallas as pl
from jax.experimental.pallas import tpu as pltpu
from jax.experimental.pallas import tpu_sc as plsc

_T = 20
_NDT = 125
_NBT = 32
_BATCH = 4096
_U = 2 * _NDT


def _build():
    info = plsc.get_sparse_core_info()
    nc = info.num_cores
    nw = nc * info.num_subcores
    mesh = plsc.VectorSubcoreMesh(core_axis_name="c", subcore_axis_name="s")

    @functools.partial(
        pl.kernel,
        mesh=mesh,
        out_type=jax.ShapeDtypeStruct((_T, _NDT, _NBT, 8, 128), jnp.float32),
        scratch_types=[
            pltpu.VMEM((10, _BATCH), jnp.int32),
            pltpu.VMEM((8000,), jnp.float32),
            pltpu.VMEM((2, _NBT, 8, 128), jnp.float32),
            pltpu.SemaphoreType.DMA,
            pltpu.SemaphoreType.DMA,
        ],
        compiler_params=pltpu.CompilerParams(
            use_tc_tiling_on_sc=False, needs_layout_passes=False),
    )
    def emb(xt_hbm, tabr_hbm, out_hbm, x_v, slice_v, blk_v, ssem, osem):
        wid = lax.axis_index("s") * nc + lax.axis_index("c")
        lo = wid * _U // nw
        hi = (wid + 1) * _U // nw

        ds_off = [jnp.full((16,), ds * 1000, jnp.int32) for ds in range(8)]

        def wait_put(slot):
            pltpu.make_async_copy(
                blk_v.at[slot],
                out_hbm.at[0, 0], osem).wait()

        def u_body(u, _):
            th = u // _NDT
            dt = lax.rem(u, _NDT)

            @pl.when(jnp.logical_or(u == lo, dt == 0))
            def _():
                pltpu.sync_copy(xt_hbm.at[pl.ds(th * 10, 10)], x_v)

            pltpu.async_copy(tabr_hbm.at[dt], slice_v, ssem).wait()

            def tp_body(tp, _):
                for sl in range(2):
                    tl = tp * 2 + sl
                    t = th * 10 + tl
                    g = (u - lo) * 10 + tl

                    @pl.when(g >= 2)
                    def _():
                        wait_put(sl)

                    @plsc.parallel_loop(0, _NBT, unroll=4)
                    def bt_body(btl):
                        for bl in range(8):
                            xv = x_v[tl, pl.ds(btl * 128 + bl * 16, 16)]
                            for ds in range(8):
                                vals = plsc.load_gather(
                                    slice_v, [xv + ds_off[ds]])
                                blk_v[sl, btl, ds, pl.ds(bl * 16, 16)] = vals

                    pltpu.async_copy(
                        blk_v.at[sl],
                        out_hbm.at[t, dt], osem)
                return 0

            lax.fori_loop(0, 5, tp_body, 0)
            return 0

        lax.fori_loop(lo, hi, u_body, 0)
        wait_put(0)
        wait_put(1)

    return emb


_emb = _build()


def kernel(x, table):
    xt = x.T.astype(jnp.int32)
    tabr = table.T.reshape(_NDT, 8000)
    out5 = _emb(xt, tabr)
    return jnp.transpose(out5, (2, 4, 0, 1, 3)).reshape(_BATCH, _T, _NDT * 8)
